# baseline (speedup 1.0000x reference)
import functools
import math

import jax
import jax.numpy as jnp
import numpy as np
from jax import lax
from jax.experimental import pallas as pl
from jax.experimental.pallas import tpu as pltpu
from jax.experimental.pallas import tpu_sc as plsc

D_Z = 128
D_H = 64
B = 16
DEPTH = 4
BUCKETS = 1 << 20
LEVEL_SIZES = [B ** l for l in range(DEPTH + 1)]
TOTAL_NONROOT = sum(LEVEL_SIZES[1:])

RB = 1024
LEVEL_OFF = [0, 1024, 2048, 6144]
N_USED = 6144 + 65536
NBLK = N_USED // RB
NW = 32
CHUNK = 128
N_PAD = 73728
CPW = N_PAD // (NW * CHUNK)
NG = N_PAD // B

_BLK_LVL = np.zeros((NBLK,), np.int32)
_BLK_LVL[1] = 1
_BLK_LVL[2:6] = 2
_BLK_LVL[6:] = 3
_ONEHOT = np.zeros((NBLK, 4), np.float32)
_ONEHOT[np.arange(NBLK), _BLK_LVL] = 1.0
_SIZES4 = np.array([[math.log1p(float(B ** (DEPTH - l)))] for l in range(1, 5)],
                   np.float32)
_PAD_IDX = (np.arange(N_PAD, dtype=np.int32) * 977) % BUCKETS


def _gather_body(flatT_hbm, idx_hbm, outT_hbm, idx_v, eidx0, eidx1, cols0,
                 cols1, gs0, gs1, os0, os1):
    wid = lax.axis_index("s") * 2 + lax.axis_index("c")
    pltpu.sync_copy(idx_hbm.at[wid], idx_v)
    base = wid * (CPW * CHUNK)
    cols = (cols0, cols1)
    eidx = (eidx0, eidx1)
    gsem = (gs0, gs1)
    osem = (os0, os1)

    def fire_chunk(c):
        buf, sem, eix = cols[c % 2], gsem[c % 2], eidx[c % 2]
        for g in range(CHUNK // 16):
            iv = idx_v[c, pl.ds(g * 16, 16)]
            eix[pl.ds(g * 16, 16)] = ((iv >> 7) << 10) + (iv & 127)

        def body(k, carry):
            off = (k // 8) * (BUCKETS * 8) + (k % 8) * 128
            pltpu.make_async_copy(
                flatT_hbm.at[pl.ds(off, BUCKETS * 8)].at[eix],
                buf.at[k], sem).start()
            return carry

        lax.fori_loop(0, D_H, body, 0)

    def drain_chunk(c):
        pltpu.make_async_copy(outT_hbm.at[:, pl.ds(0, CHUNK)], cols[c % 2],
                              gsem[c % 2]).wait()

    oh = [None] * CPW
    fire_chunk(0)
    for c in range(CPW):
        nxt = c + 1
        if nxt < CPW:
            if c >= 1:
                oh[c - 1].wait()
            fire_chunk(nxt)
        drain_chunk(c)
        oh[c] = pltpu.async_copy(
            cols[c % 2], outT_hbm.at[:, pl.ds(base + c * CHUNK, CHUNK)],
            osem[c % 2])
    oh[CPW - 2].wait()
    oh[CPW - 1].wait()


@functools.cache
def _make_gather():
    return pl.kernel(
        _gather_body,
        out_type=jax.ShapeDtypeStruct((D_H, N_PAD), jnp.float32),
        mesh=plsc.VectorSubcoreMesh(core_axis_name="c", subcore_axis_name="s"),
        compiler_params=pltpu.CompilerParams(use_tc_tiling_on_sc=True),
        scratch_types=[
            pltpu.VMEM((CPW, CHUNK), jnp.int32),
            pltpu.VMEM((CHUNK,), jnp.int32),
            pltpu.VMEM((CHUNK,), jnp.int32),
            pltpu.VMEM((D_H, CHUNK), jnp.float32),
            pltpu.VMEM((D_H, CHUNK), jnp.float32),
            pltpu.SemaphoreType.DMA,
            pltpu.SemaphoreType.DMA,
            pltpu.SemaphoreType.DMA,
            pltpu.SemaphoreType.DMA,
        ],
    )


def _gather(tableT, idx3):
    flat_phys = tableT.reshape(8, 8, BUCKETS // 128, 128)
    flat_phys = flat_phys.transpose(0, 2, 1, 3).reshape(-1)
    return _make_gather()(flat_phys, idx3)


def _prep_body(z2, rW1T, rb1r, rW2r, rb2r, cW1zT, cW1lT, cW1sr, cb1r, lt4,
               sizes4, onehot, root_o, bias_o):
    h = jnp.maximum(jnp.dot(z2[...], rW1T[...],
                            preferred_element_type=jnp.float32) + rb1r[...], 0.0)
    x = jnp.sum(h * rW2r[...], axis=1, keepdims=True) + rb2r[...]
    root_o[...] = jnp.maximum(x, 0.0) + jnp.log(1.0 + jnp.exp(-jnp.abs(x)))
    bias4 = (jnp.dot(z2[...], cW1zT[...], preferred_element_type=jnp.float32)
             + jnp.dot(lt4[...], cW1lT[...], preferred_element_type=jnp.float32)
             + sizes4[...] * cW1sr[...] + cb1r[...])
    bias_o[...] = jnp.dot(onehot[...], bias4, preferred_element_type=jnp.float32)


def _run_prep(z2, rW1T, rb1r, rW2r, rb2r, cW1zT, cW1lT, cW1sr, cb1r, lt4):
    return pl.pallas_call(
        _prep_body,
        out_shape=[
            jax.ShapeDtypeStruct((1, 1), jnp.float32),
            jax.ShapeDtypeStruct((NBLK, D_H), jnp.float32),
        ],
    )(z2, rW1T, rb1r, rW2r, rb2r, cW1zT, cW1lT, cW1sr, cb1r, lt4,
      jnp.asarray(_SIZES4), jnp.asarray(_ONEHOT))


def _score_body(peT, bias, w1, w2, logit_o):
    hhT = jnp.maximum(jnp.dot(w1[...], peT[...],
                              preferred_element_type=jnp.float32)
                      + bias[...].reshape(D_H, 1), 0.0)
    logit_o[...] = jnp.dot(w2[...], hhT, preferred_element_type=jnp.float32)


def _run_score(peT_pad, bias3, w1, w2):
    return pl.pallas_call(
        _score_body,
        grid=(NBLK,),
        in_specs=[
            pl.BlockSpec((D_H, RB), lambda i: (0, i)),
            pl.BlockSpec((1, D_H, 1), lambda i: (i, 0, 0)),
            pl.BlockSpec((D_H, D_H), lambda i: (0, 0)),
            pl.BlockSpec((1, D_H), lambda i: (0, 0)),
        ],
        out_specs=pl.BlockSpec((1, RB), lambda i: (0, i)),
        out_shape=jax.ShapeDtypeStruct((1, N_PAD), jnp.float32),
    )(peT_pad, bias3, w1, w2)


def _softmax16(x):
    m = jnp.max(x, axis=1, keepdims=True)
    e = jnp.exp(x - m)
    return e / jnp.sum(e, axis=1, keepdims=True)


def _mass_body(lg, root, m1_o, m2_o, m3_o, m4_o):
    c = root[...]
    m1 = _softmax16(lg[0:1, :]) * c
    m1_o[...] = m1
    a2 = _softmax16(lg[64:80, :])
    m2 = (a2.reshape(1, B, B) * m1[:, :, None]).reshape(B, B)
    m2_o[...] = m2
    a3 = _softmax16(lg[128:384, :])
    m3 = (a3.reshape(B, B, B) * m2[:, :, None]).reshape(B * B, B)
    m3_o[...] = m3
    a4 = _softmax16(lg[384:4480, :])
    m4_o[...] = (a4.reshape(B * B, B, B) * m3[:, :, None]).reshape(B ** 3, B)


def _run_mass(lg, root):
    return pl.pallas_call(
        _mass_body,
        out_shape=[
            jax.ShapeDtypeStruct((1, B), jnp.float32),
            jax.ShapeDtypeStruct((B, B), jnp.float32),
            jax.ShapeDtypeStruct((B * B, B), jnp.float32),
            jax.ShapeDtypeStruct((B ** 3, B), jnp.float32),
        ],
    )(lg, root)


def kernel(z_c, prefix_hash, prefix_table, level_table, rW1, rb1, rW2, rb2,
           cW1, cb1, cW2, cb2):
    ph = prefix_hash.astype(jnp.int32)
    idx_pad = jnp.asarray(_PAD_IDX)
    off = 0
    for l in range(4):
        n = LEVEL_SIZES[l + 1]
        idx_pad = idx_pad.at[LEVEL_OFF[l]:LEVEL_OFF[l] + n].set(ph[off:off + n])
        off += n
    idx3 = idx_pad.reshape(NW, CPW, CHUNK)

    peT_pad = _gather(prefix_table.T, idx3)

    z2 = z_c.reshape(1, D_Z)
    root, bias = _run_prep(
        z2, rW1.T, rb1.reshape(1, D_H), rW2.reshape(1, D_H),
        rb2.reshape(1, 1), cW1[:, :D_Z].T, cW1[:, D_Z + D_H:D_Z + 2 * D_H].T,
        cW1[:, D_Z + 2 * D_H].reshape(1, D_H), cb1.reshape(1, D_H),
        level_table[1:5])

    logits = _run_score(peT_pad, bias.reshape(NBLK, D_H, 1),
                        cW1[:, D_Z:D_Z + D_H], cW2.reshape(1, D_H))

    m1, m2, m3, m4 = _run_mass(logits.reshape(NG, B), root)
    m4f = m4.reshape(-1)
    return (m4f, root.reshape(1), m1.reshape(-1), m2.reshape(-1),
            m3.reshape(-1), m4f)


# speedup vs baseline: 1.0908x; 1.0908x over previous
import functools
import math

import jax
import jax.numpy as jnp
import numpy as np
from jax import lax
from jax.experimental import pallas as pl
from jax.experimental.pallas import tpu as pltpu
from jax.experimental.pallas import tpu_sc as plsc

D_Z = 128
D_H = 64
B = 16
DEPTH = 4
BUCKETS = 1 << 20
LEVEL_SIZES = [B ** l for l in range(DEPTH + 1)]
TOTAL_NONROOT = sum(LEVEL_SIZES[1:])

RB = 1024
LEVEL_OFF = [0, 1024, 2048, 6144]
N_USED = 6144 + 65536
NBLK = N_USED // RB
NW = 32
CHUNK = 128
N_PAD = 73728
CPW = N_PAD // (NW * CHUNK)
NG = N_PAD // B
HALF_CPW = CPW // 2
HALF_N = NW * HALF_CPW * CHUNK
NBLK_A = HALF_N // RB
NBLK_B = NBLK - NBLK_A
HGROUPS = HALF_N // B

_BLK_LVL = np.zeros((NBLK,), np.int32)
_BLK_LVL[1] = 1
_BLK_LVL[2:6] = 2
_BLK_LVL[6:] = 3
_ONEHOT = np.zeros((NBLK, 4), np.float32)
_ONEHOT[np.arange(NBLK), _BLK_LVL] = 1.0
_SIZES4 = np.array([[math.log1p(float(B ** (DEPTH - l)))] for l in range(1, 5)],
                   np.float32)
_PAD_IDX = (np.arange(N_PAD, dtype=np.int32) * 977) % BUCKETS


def _gather_body(cpw, flatT_hbm, idx_hbm, outT_hbm, idx_v, eidx0, eidx1,
                 cols0, cols1, gs0, gs1, os0, os1):
    wid = lax.axis_index("s") * 2 + lax.axis_index("c")
    pltpu.sync_copy(idx_hbm.at[wid], idx_v)
    base = wid * (cpw * CHUNK)
    cols = (cols0, cols1)
    eidx = (eidx0, eidx1)
    gsem = (gs0, gs1)
    osem = (os0, os1)

    def fire_chunk(c):
        buf, sem, eix = cols[c % 2], gsem[c % 2], eidx[c % 2]
        for g in range(CHUNK // 16):
            iv = idx_v[c, pl.ds(g * 16, 16)]
            eix[pl.ds(g * 16, 16)] = ((iv >> 7) << 10) + (iv & 127)

        def body(k, carry):
            off = (k // 8) * (BUCKETS * 8) + (k % 8) * 128
            pltpu.make_async_copy(
                flatT_hbm.at[pl.ds(off, BUCKETS * 8)].at[eix],
                buf.at[k], sem).start()
            return carry

        lax.fori_loop(0, D_H, body, 0)

    def drain_chunk(c):
        pltpu.make_async_copy(outT_hbm.at[:, pl.ds(0, CHUNK)], cols[c % 2],
                              gsem[c % 2]).wait()

    oh = [None] * cpw
    fire_chunk(0)
    for c in range(cpw):
        nxt = c + 1
        if nxt < cpw:
            if c >= 1:
                oh[c - 1].wait()
            fire_chunk(nxt)
        drain_chunk(c)
        oh[c] = pltpu.async_copy(
            cols[c % 2], outT_hbm.at[:, pl.ds(base + c * CHUNK, CHUNK)],
            osem[c % 2])
    oh[cpw - 2].wait()
    oh[cpw - 1].wait()


@functools.cache
def _make_gather(cpw):
    return pl.kernel(
        functools.partial(_gather_body, cpw),
        out_type=jax.ShapeDtypeStruct((D_H, NW * cpw * CHUNK), jnp.float32),
        mesh=plsc.VectorSubcoreMesh(core_axis_name="c", subcore_axis_name="s"),
        compiler_params=pltpu.CompilerParams(use_tc_tiling_on_sc=True),
        scratch_types=[
            pltpu.VMEM((cpw, CHUNK), jnp.int32),
            pltpu.VMEM((CHUNK,), jnp.int32),
            pltpu.VMEM((CHUNK,), jnp.int32),
            pltpu.VMEM((D_H, CHUNK), jnp.float32),
            pltpu.VMEM((D_H, CHUNK), jnp.float32),
            pltpu.SemaphoreType.DMA,
            pltpu.SemaphoreType.DMA,
            pltpu.SemaphoreType.DMA,
            pltpu.SemaphoreType.DMA,
        ],
    )


def _gather(tableT, idx3):
    flat_phys = tableT.reshape(8, 8, BUCKETS // 128, 128)
    flat_phys = flat_phys.transpose(0, 2, 1, 3).reshape(-1)
    return _make_gather(idx3.shape[1])(flat_phys, idx3)


def _prep_body(z2, rW1T, rb1r, rW2r, rb2r, cW1zT, cW1lT, cW1sr, cb1r, lt4,
               sizes4, onehot, root_o, bias_o):
    h = jnp.maximum(jnp.dot(z2[...], rW1T[...],
                            preferred_element_type=jnp.float32) + rb1r[...], 0.0)
    x = jnp.sum(h * rW2r[...], axis=1, keepdims=True) + rb2r[...]
    root_o[...] = jnp.maximum(x, 0.0) + jnp.log(1.0 + jnp.exp(-jnp.abs(x)))
    bias4 = (jnp.dot(z2[...], cW1zT[...], preferred_element_type=jnp.float32)
             + jnp.dot(lt4[...], cW1lT[...], preferred_element_type=jnp.float32)
             + sizes4[...] * cW1sr[...] + cb1r[...])
    bias_o[...] = jnp.dot(onehot[...], bias4, preferred_element_type=jnp.float32)


def _run_prep(z2, rW1T, rb1r, rW2r, rb2r, cW1zT, cW1lT, cW1sr, cb1r, lt4):
    return pl.pallas_call(
        _prep_body,
        out_shape=[
            jax.ShapeDtypeStruct((1, 1), jnp.float32),
            jax.ShapeDtypeStruct((NBLK, D_H), jnp.float32),
        ],
    )(z2, rW1T, rb1r, rW2r, rb2r, cW1zT, cW1lT, cW1sr, cb1r, lt4,
      jnp.asarray(_SIZES4), jnp.asarray(_ONEHOT))


def _score_body(peT, bias, w1, w2, logit_o):
    hhT = jnp.maximum(jnp.dot(w1[...], peT[...],
                              preferred_element_type=jnp.float32)
                      + bias[...].reshape(D_H, 1), 0.0)
    logit_o[...] = jnp.dot(w2[...], hhT, preferred_element_type=jnp.float32)


def _run_score(peT_pad, bias3, w1, w2):
    nblk = bias3.shape[0]
    width = peT_pad.shape[1]
    return pl.pallas_call(
        _score_body,
        grid=(nblk,),
        in_specs=[
            pl.BlockSpec((D_H, RB), lambda i: (0, i)),
            pl.BlockSpec((1, D_H, 1), lambda i: (i, 0, 0)),
            pl.BlockSpec((D_H, D_H), lambda i: (0, 0)),
            pl.BlockSpec((1, D_H), lambda i: (0, 0)),
        ],
        out_specs=pl.BlockSpec((1, RB), lambda i: (0, i)),
        out_shape=jax.ShapeDtypeStruct((1, width), jnp.float32),
    )(peT_pad, bias3, w1, w2)


def _softmax16(x):
    m = jnp.max(x, axis=1, keepdims=True)
    e = jnp.exp(x - m)
    return e / jnp.sum(e, axis=1, keepdims=True)


def _mass_body(lga, lgb, root, m1_o, m2_o, m3_o, m4_o):
    c = root[...]
    m1 = _softmax16(lga[0:1, :]) * c
    m1_o[...] = m1
    a2 = _softmax16(lga[64:80, :])
    m2 = (a2.reshape(1, B, B) * m1[:, :, None]).reshape(B, B)
    m2_o[...] = m2
    a3 = _softmax16(lga[128:384, :])
    m3 = (a3.reshape(B, B, B) * m2[:, :, None]).reshape(B * B, B)
    m3_o[...] = m3
    lg4 = jnp.concatenate([lga[384:HGROUPS, :], lgb[0:4480 - HGROUPS, :]],
                          axis=0)
    a4 = _softmax16(lg4)
    m4_o[...] = (a4.reshape(B * B, B, B) * m3[:, :, None]).reshape(B ** 3, B)


def _run_mass(lga, lgb, root):
    return pl.pallas_call(
        _mass_body,
        out_shape=[
            jax.ShapeDtypeStruct((1, B), jnp.float32),
            jax.ShapeDtypeStruct((B, B), jnp.float32),
            jax.ShapeDtypeStruct((B * B, B), jnp.float32),
            jax.ShapeDtypeStruct((B ** 3, B), jnp.float32),
        ],
    )(lga, lgb, root)


def kernel(z_c, prefix_hash, prefix_table, level_table, rW1, rb1, rW2, rb2,
           cW1, cb1, cW2, cb2):
    ph = prefix_hash.astype(jnp.int32)
    idx_pad = jnp.asarray(_PAD_IDX)
    off = 0
    for l in range(4):
        n = LEVEL_SIZES[l + 1]
        idx_pad = idx_pad.at[LEVEL_OFF[l]:LEVEL_OFF[l] + n].set(ph[off:off + n])
        off += n
    idxA = idx_pad[:HALF_N].reshape(NW, HALF_CPW, CHUNK)
    idxB = idx_pad[HALF_N:].reshape(NW, HALF_CPW, CHUNK)

    tT = prefix_table.T
    peA = _gather(tT, idxA)
    peB = _gather(tT, idxB)

    z2 = z_c.reshape(1, D_Z)
    root, bias = _run_prep(
        z2, rW1.T, rb1.reshape(1, D_H), rW2.reshape(1, D_H),
        rb2.reshape(1, 1), cW1[:, :D_Z].T, cW1[:, D_Z + D_H:D_Z + 2 * D_H].T,
        cW1[:, D_Z + 2 * D_H].reshape(1, D_H), cb1.reshape(1, D_H),
        level_table[1:5])

    bias3 = bias.reshape(NBLK, D_H, 1)
    w1 = cW1[:, D_Z:D_Z + D_H]
    w2 = cW2.reshape(1, D_H)
    logitsA = _run_score(peA, bias3[:NBLK_A], w1, w2)
    logitsB = _run_score(peB, bias3[NBLK_A:NBLK], w1, w2)

    m1, m2, m3, m4 = _run_mass(logitsA.reshape(HGROUPS, B),
                               logitsB.reshape(HGROUPS, B), root)
    m4f = m4.reshape(-1)
    return (m4f, root.reshape(1), m1.reshape(-1), m2.reshape(-1),
            m3.reshape(-1), m4f)


# speedup vs baseline: 1.1089x; 1.0167x over previous
import functools
import math

import jax
import jax.numpy as jnp
import numpy as np
from jax import lax
from jax.experimental import pallas as pl
from jax.experimental.pallas import tpu as pltpu
from jax.experimental.pallas import tpu_sc as plsc

D_Z = 128
D_H = 64
B = 16
DEPTH = 4
BUCKETS = 1 << 20
LEVEL_SIZES = [B ** l for l in range(DEPTH + 1)]
TOTAL_NONROOT = sum(LEVEL_SIZES[1:])

RB = 1024
LEVEL_OFF = [0, 1024, 2048, 6144]
N_USED = 6144 + 65536
NBLK = N_USED // RB
NW = 32
CHUNK = 128
N_PAD = 73728
CPW = N_PAD // (NW * CHUNK)
NG = N_PAD // B
PIECES = [10, 4, 4]
PIECE_N = [NW * c * CHUNK for c in PIECES]
PIECE_G = [n // B for n in PIECE_N]

_BLK_LVL = np.zeros((NBLK,), np.int32)
_BLK_LVL[1] = 1
_BLK_LVL[2:6] = 2
_BLK_LVL[6:] = 3
_ONEHOT = np.zeros((NBLK, 4), np.float32)
_ONEHOT[np.arange(NBLK), _BLK_LVL] = 1.0
_SIZES4 = np.array([[math.log1p(float(B ** (DEPTH - l)))] for l in range(1, 5)],
                   np.float32)
_PAD_IDX = (np.arange(N_PAD, dtype=np.int32) * 977) % BUCKETS


def _gather_body(cpw, flatT_hbm, idx_hbm, outT_hbm, idx_v, eidx0, eidx1,
                 cols0, cols1, gs0, gs1, os0, os1):
    wid = lax.axis_index("s") * 2 + lax.axis_index("c")
    pltpu.sync_copy(idx_hbm.at[wid], idx_v)
    base = wid * (cpw * CHUNK)
    cols = (cols0, cols1)
    eidx = (eidx0, eidx1)
    gsem = (gs0, gs1)
    osem = (os0, os1)

    def fire_chunk(c):
        buf, sem, eix = cols[c % 2], gsem[c % 2], eidx[c % 2]
        for g in range(CHUNK // 16):
            iv = idx_v[c, pl.ds(g * 16, 16)]
            eix[pl.ds(g * 16, 16)] = ((iv >> 7) << 10) + (iv & 127)

        def body(k, carry):
            off = (k // 8) * (BUCKETS * 8) + (k % 8) * 128
            pltpu.make_async_copy(
                flatT_hbm.at[pl.ds(off, BUCKETS * 8)].at[eix],
                buf.at[k], sem).start()
            return carry

        lax.fori_loop(0, D_H, body, 0)

    def drain_chunk(c):
        pltpu.make_async_copy(outT_hbm.at[:, pl.ds(0, CHUNK)], cols[c % 2],
                              gsem[c % 2]).wait()

    oh = [None] * cpw
    fire_chunk(0)
    for c in range(cpw):
        nxt = c + 1
        if nxt < cpw:
            if c >= 1:
                oh[c - 1].wait()
            fire_chunk(nxt)
        drain_chunk(c)
        oh[c] = pltpu.async_copy(
            cols[c % 2], outT_hbm.at[:, pl.ds(base + c * CHUNK, CHUNK)],
            osem[c % 2])
    oh[cpw - 2].wait()
    oh[cpw - 1].wait()


@functools.cache
def _make_gather(cpw):
    return pl.kernel(
        functools.partial(_gather_body, cpw),
        out_type=jax.ShapeDtypeStruct((D_H, NW * cpw * CHUNK), jnp.float32),
        mesh=plsc.VectorSubcoreMesh(core_axis_name="c", subcore_axis_name="s"),
        compiler_params=pltpu.CompilerParams(use_tc_tiling_on_sc=True),
        scratch_types=[
            pltpu.VMEM((cpw, CHUNK), jnp.int32),
            pltpu.VMEM((CHUNK,), jnp.int32),
            pltpu.VMEM((CHUNK,), jnp.int32),
            pltpu.VMEM((D_H, CHUNK), jnp.float32),
            pltpu.VMEM((D_H, CHUNK), jnp.float32),
            pltpu.SemaphoreType.DMA,
            pltpu.SemaphoreType.DMA,
            pltpu.SemaphoreType.DMA,
            pltpu.SemaphoreType.DMA,
        ],
    )


def _gather(tableT, idx3):
    flat_phys = tableT.reshape(8, 8, BUCKETS // 128, 128)
    flat_phys = flat_phys.transpose(0, 2, 1, 3).reshape(-1)
    return _make_gather(idx3.shape[1])(flat_phys, idx3)


def _prep_body(z2, rW1T, rb1r, rW2r, rb2r, cW1zT, cW1lT, cW1sr, cb1r, lt4,
               sizes4, onehot, root_o, bias_o):
    h = jnp.maximum(jnp.dot(z2[...], rW1T[...],
                            preferred_element_type=jnp.float32) + rb1r[...], 0.0)
    x = jnp.sum(h * rW2r[...], axis=1, keepdims=True) + rb2r[...]
    root_o[...] = jnp.maximum(x, 0.0) + jnp.log(1.0 + jnp.exp(-jnp.abs(x)))
    bias4 = (jnp.dot(z2[...], cW1zT[...], preferred_element_type=jnp.float32)
             + jnp.dot(lt4[...], cW1lT[...], preferred_element_type=jnp.float32)
             + sizes4[...] * cW1sr[...] + cb1r[...])
    bias_o[...] = jnp.dot(onehot[...], bias4, preferred_element_type=jnp.float32)


def _run_prep(z2, rW1T, rb1r, rW2r, rb2r, cW1zT, cW1lT, cW1sr, cb1r, lt4):
    return pl.pallas_call(
        _prep_body,
        out_shape=[
            jax.ShapeDtypeStruct((1, 1), jnp.float32),
            jax.ShapeDtypeStruct((NBLK, D_H), jnp.float32),
        ],
    )(z2, rW1T, rb1r, rW2r, rb2r, cW1zT, cW1lT, cW1sr, cb1r, lt4,
      jnp.asarray(_SIZES4), jnp.asarray(_ONEHOT))


def _score_body(peT, bias, w1, w2, logit_o):
    hhT = jnp.maximum(jnp.dot(w1[...], peT[...],
                              preferred_element_type=jnp.float32)
                      + bias[...].reshape(D_H, 1), 0.0)
    logit_o[...] = jnp.dot(w2[...], hhT, preferred_element_type=jnp.float32)


def _run_score(peT_pad, bias3, w1, w2):
    nblk = bias3.shape[0]
    width = peT_pad.shape[1]
    return pl.pallas_call(
        _score_body,
        grid=(nblk,),
        in_specs=[
            pl.BlockSpec((D_H, RB), lambda i: (0, i)),
            pl.BlockSpec((1, D_H, 1), lambda i: (i, 0, 0)),
            pl.BlockSpec((D_H, D_H), lambda i: (0, 0)),
            pl.BlockSpec((1, D_H), lambda i: (0, 0)),
        ],
        out_specs=pl.BlockSpec((1, RB), lambda i: (0, i)),
        out_shape=jax.ShapeDtypeStruct((1, width), jnp.float32),
    )(peT_pad, bias3, w1, w2)


def _softmax16(x):
    m = jnp.max(x, axis=1, keepdims=True)
    e = jnp.exp(x - m)
    return e / jnp.sum(e, axis=1, keepdims=True)


def _mass_body(lga, lgb, lgc, root, m1_o, m2_o, m3_o, m4_o):
    c = root[...]
    m1 = _softmax16(lga[0:1, :]) * c
    m1_o[...] = m1
    a2 = _softmax16(lga[64:80, :])
    m2 = (a2.reshape(1, B, B) * m1[:, :, None]).reshape(B, B)
    m2_o[...] = m2
    a3 = _softmax16(lga[128:384, :])
    m3 = (a3.reshape(B, B, B) * m2[:, :, None]).reshape(B * B, B)
    m3_o[...] = m3
    ga, gb = PIECE_G[0], PIECE_G[1]
    lg4 = jnp.concatenate(
        [lga[384:ga, :], lgb[...], lgc[0:4480 - ga - gb, :]],
        axis=0)
    a4 = _softmax16(lg4)
    m4_o[...] = (a4.reshape(B * B, B, B) * m3[:, :, None]).reshape(B ** 3, B)


def _run_mass(lga, lgb, lgc, root):
    return pl.pallas_call(
        _mass_body,
        out_shape=[
            jax.ShapeDtypeStruct((1, B), jnp.float32),
            jax.ShapeDtypeStruct((B, B), jnp.float32),
            jax.ShapeDtypeStruct((B * B, B), jnp.float32),
            jax.ShapeDtypeStruct((B ** 3, B), jnp.float32),
        ],
    )(lga, lgb, lgc, root)


def kernel(z_c, prefix_hash, prefix_table, level_table, rW1, rb1, rW2, rb2,
           cW1, cb1, cW2, cb2):
    ph = prefix_hash.astype(jnp.int32)
    idx_pad = jnp.asarray(_PAD_IDX)
    off = 0
    for l in range(4):
        n = LEVEL_SIZES[l + 1]
        idx_pad = idx_pad.at[LEVEL_OFF[l]:LEVEL_OFF[l] + n].set(ph[off:off + n])
        off += n
    tT = prefix_table.T
    pes = []
    node0 = 0
    for cpw, n in zip(PIECES, PIECE_N):
        pes.append(_gather(tT, idx_pad[node0:node0 + n].reshape(
            NW, cpw, CHUNK)))
        node0 += n

    z2 = z_c.reshape(1, D_Z)
    root, bias = _run_prep(
        z2, rW1.T, rb1.reshape(1, D_H), rW2.reshape(1, D_H),
        rb2.reshape(1, 1), cW1[:, :D_Z].T, cW1[:, D_Z + D_H:D_Z + 2 * D_H].T,
        cW1[:, D_Z + 2 * D_H].reshape(1, D_H), cb1.reshape(1, D_H),
        level_table[1:5])

    bias3 = bias.reshape(NBLK, D_H, 1)
    w1 = cW1[:, D_Z:D_Z + D_H]
    w2 = cW2.reshape(1, D_H)
    lgs = []
    blk0 = 0
    for pe, g in zip(pes, PIECE_G):
        nblk = min(pe.shape[1] // RB, NBLK - blk0)
        lgs.append(_run_score(pe, bias3[blk0:blk0 + nblk], w1, w2)
                   .reshape(g, B))
        blk0 += nblk

    m1, m2, m3, m4 = _run_mass(lgs[0], lgs[1], lgs[2], root)
    m4f = m4.reshape(-1)
    return (m4f, root.reshape(1), m1.reshape(-1), m2.reshape(-1),
            m3.reshape(-1), m4f)
